# baseline (speedup 1.0000x reference)
import jax
import jax.numpy as jnp
from jax import lax
from jax.experimental import pallas as pl
from jax.experimental.pallas import tpu as pltpu
from jax.experimental.pallas import tpu_sc as plsc

N = 10000
E = 320000
F_IN = 128
H = 64
G = 128

NC = 2
NS = 16
NW = NC * NS
E_PER_TILE = E // NW
K = 80
CHUNKS = E_PER_TILE // K
N_PAD = 10240
ROWS_PER_TILE = N_PAD // NS
ZROWS = 128

CNT_W = 16


def _mesh():
    return plsc.VectorSubcoreMesh(
        core_axis_name="c", subcore_axis_name="s",
        num_cores=NC, num_subcores=NS)


_SC_PARAMS = pltpu.CompilerParams(use_tc_tiling_on_sc=False)


def _zero_fill(buf, width):
    def body(i, carry):
        for c in range(width // 16):
            buf[i, pl.ds(c * 16, 16)] = jnp.zeros((16,), jnp.float32)
        return carry
    lax.fori_loop(0, ZROWS, body, None)


def _zero_my_slice(zbuf, acc_sh, sid):
    for r in range(ROWS_PER_TILE // ZROWS):
        pltpu.sync_copy(zbuf, acc_sh.at[pl.ds(sid * ROWS_PER_TILE + r * ZROWS, ZROWS)])


NB = 5
GROUPS = CHUNKS // NB


def _sc_count_body(dst_hbm, out_hbm, didx_v, sdidx_v, ones_v, zbuf_v,
                   sem_i, sem_s, acc_sh):
    cid = lax.axis_index("c")
    sid = lax.axis_index("s")

    def fill_ones(i, carry):
        ones_v[i, :] = jnp.ones((16,), jnp.float32)
        return carry
    lax.fori_loop(0, K, fill_ones, None)
    _zero_fill(zbuf_v, CNT_W)
    _zero_my_slice(zbuf_v, acc_sh, sid)
    plsc.subcore_barrier()

    base = cid * (E // NC) + sid * E_PER_TILE

    def start_idx(j, b):
        pltpu.async_copy(dst_hbm.at[pl.ds(base + j * K, K)], didx_v.at[b],
                         sem_i.at[b])

    def wait_idx(b):
        pltpu.make_async_copy(dst_hbm.at[pl.ds(0, K)], didx_v.at[b],
                              sem_i.at[b]).wait()

    def wait_scat(b):
        pltpu.make_async_copy(ones_v, acc_sh.at[sdidx_v.at[b]],
                              sem_s.at[b]).wait()

    for b in range(NB):
        start_idx(b, b)

    def group(g, carry):
        for b in range(NB):
            j = g * NB + b
            wait_idx(b)

            @pl.when(g > 0)
            def _():
                wait_scat(b)
            for w in range(K // 16):
                sdidx_v[b, pl.ds(w * 16, 16)] = didx_v[b, pl.ds(w * 16, 16)]
            pltpu.async_copy(ones_v, acc_sh.at[sdidx_v.at[b]], sem_s.at[b],
                             add=True)

            @pl.when(j + NB < CHUNKS)
            def _():
                start_idx(j + NB, b)
        return carry
    lax.fori_loop(0, GROUPS, group, None)
    for b in range(NB):
        wait_scat(b)
    plsc.subcore_barrier()

    pltpu.sync_copy(
        acc_sh.at[pl.ds(sid * ROWS_PER_TILE, ROWS_PER_TILE)],
        out_hbm.at[pl.ds(cid * N_PAD + sid * ROWS_PER_TILE, ROWS_PER_TILE)])


_sc_count = pl.kernel(
    _sc_count_body,
    out_type=jax.ShapeDtypeStruct((NC * N_PAD, CNT_W), jnp.float32),
    mesh=_mesh(),
    scratch_types=[
        pltpu.VMEM((NB, K), jnp.int32),
        pltpu.VMEM((NB, K), jnp.int32),
        pltpu.VMEM((K, CNT_W), jnp.float32),
        pltpu.VMEM((ZROWS, CNT_W), jnp.float32),
        pltpu.SemaphoreType.DMA((NB,)),
        pltpu.SemaphoreType.DMA((NB,)),
        pltpu.VMEM_SHARED((N_PAD, CNT_W), jnp.float32),
    ],
    compiler_params=_SC_PARAMS,
)


def _sc_agg_body(src_hbm, dst_hbm, y_hbm, out_hbm,
                 sidx_v, didx_v, ssidx_v, sdidx_v, rows_v, zbuf_v,
                 sem_i, sem_g, sem_s, acc_sh):
    cid = lax.axis_index("c")
    sid = lax.axis_index("s")

    _zero_fill(zbuf_v, H)
    _zero_my_slice(zbuf_v, acc_sh, sid)
    plsc.subcore_barrier()

    base = cid * (E // NC) + sid * E_PER_TILE

    def start_idx(j, b):
        off = base + j * K
        pltpu.async_copy(src_hbm.at[pl.ds(off, K)], sidx_v.at[b], sem_i.at[b])
        pltpu.async_copy(dst_hbm.at[pl.ds(off, K)], didx_v.at[b], sem_i.at[b])

    def wait_idx(b):
        pltpu.make_async_copy(src_hbm.at[pl.ds(0, K)], sidx_v.at[b], sem_i.at[b]).wait()
        pltpu.make_async_copy(dst_hbm.at[pl.ds(0, K)], didx_v.at[b], sem_i.at[b]).wait()

    def wait_gather(b):
        pltpu.make_async_copy(y_hbm.at[ssidx_v.at[b]], rows_v.at[b],
                              sem_g.at[b]).wait()

    def wait_scat(b):
        pltpu.make_async_copy(rows_v.at[b], acc_sh.at[sdidx_v.at[b]],
                              sem_s.at[b]).wait()

    for b in range(NB):
        start_idx(b, b)

    def group(g, carry):
        for b in range(NB):
            j = g * NB + b
            wait_idx(b)

            @pl.when(g > 0)
            def _():
                wait_scat(b)
            for w in range(K // 16):
                ssidx_v[b, pl.ds(w * 16, 16)] = sidx_v[b, pl.ds(w * 16, 16)]
                sdidx_v[b, pl.ds(w * 16, 16)] = didx_v[b, pl.ds(w * 16, 16)]
            pltpu.async_copy(y_hbm.at[ssidx_v.at[b]], rows_v.at[b], sem_g.at[b])

            @pl.when(j + NB < CHUNKS)
            def _():
                start_idx(j + NB, b)
            wait_gather(b)
            pltpu.async_copy(rows_v.at[b], acc_sh.at[sdidx_v.at[b]],
                             sem_s.at[b], add=True)
        return carry
    lax.fori_loop(0, GROUPS, group, None)
    for b in range(NB):
        wait_scat(b)
    plsc.subcore_barrier()

    pltpu.sync_copy(
        acc_sh.at[pl.ds(sid * ROWS_PER_TILE, ROWS_PER_TILE)],
        out_hbm.at[pl.ds(cid * N_PAD + sid * ROWS_PER_TILE, ROWS_PER_TILE)])


_sc_agg = pl.kernel(
    _sc_agg_body,
    out_type=jax.ShapeDtypeStruct((NC * N_PAD, H), jnp.float32),
    mesh=_mesh(),
    scratch_types=[
        pltpu.VMEM((NB, K), jnp.int32),
        pltpu.VMEM((NB, K), jnp.int32),
        pltpu.VMEM((NB, K), jnp.int32),
        pltpu.VMEM((NB, K), jnp.int32),
        pltpu.VMEM((NB, K, H), jnp.float32),
        pltpu.VMEM((ZROWS, H), jnp.float32),
        pltpu.SemaphoreType.DMA((NB,)),
        pltpu.SemaphoreType.DMA((NB,)),
        pltpu.SemaphoreType.DMA((NB,)),
        pltpu.VMEM_SHARED((N_PAD, H), jnp.float32),
    ],
    compiler_params=_SC_PARAMS,
)


def _tc_first_body(cnt_ref, x_ref, w1_ref, y1_ref, dinv_ref):
    cnt = cnt_ref[...]
    deg = cnt[0:N, :] + cnt[N_PAD:N_PAD + N, :] + 1.0
    d16 = lax.rsqrt(deg)
    d64 = jnp.concatenate([d16] * (H // CNT_W), axis=1)
    t = jnp.dot(x_ref[...], w1_ref[...], preferred_element_type=jnp.float32)
    y1_ref[...] = t * d64
    dinv_ref[...] = d64


_tc_first = pl.pallas_call(
    _tc_first_body,
    out_shape=[
        jax.ShapeDtypeStruct((N, H), jnp.float32),
        jax.ShapeDtypeStruct((N, H), jnp.float32),
    ],
)


def _tc_mid_body(agg_ref, y1_ref, dinv_ref, b1_ref, w2_ref, y2_ref):
    agg = agg_ref[0:N, :] + agg_ref[N_PAD:N_PAD + N, :] + y1_ref[...]
    d = dinv_ref[...]
    h = jnp.maximum(d * agg + b1_ref[...], 0.0)
    y2_ref[...] = jnp.dot(h, w2_ref[...], preferred_element_type=jnp.float32) * d


_tc_mid = pl.pallas_call(
    _tc_mid_body,
    out_shape=jax.ShapeDtypeStruct((N, H), jnp.float32),
)


def _tc_head_body(agg_ref, y2_ref, dinv_ref, b2_ref, batch_ref,
                  wh1_ref, bh1_ref, wh2_ref, bh2_ref, out_ref):
    agg = agg_ref[0:N, :] + agg_ref[N_PAD:N_PAD + N, :] + y2_ref[...]
    h = jnp.maximum(dinv_ref[...] * agg + b2_ref[...], 0.0)
    seg = lax.broadcasted_iota(jnp.int32, (G, N), 0)
    mask = (seg == batch_ref[...][None, :]).astype(jnp.float32)
    sums = jnp.dot(mask, h, preferred_element_type=jnp.float32)
    cnt = jnp.sum(mask, axis=1, keepdims=True)
    pooled = sums / jnp.maximum(cnt, 1.0)
    z = jnp.maximum(
        jnp.dot(pooled, wh1_ref[...], preferred_element_type=jnp.float32)
        + bh1_ref[...], 0.0)
    out_ref[...] = (jnp.dot(z, wh2_ref[...], preferred_element_type=jnp.float32)
                    + bh2_ref[...])


_tc_head = pl.pallas_call(
    _tc_head_body,
    out_shape=jax.ShapeDtypeStruct((G, 2), jnp.float32),
)


def kernel(x, edge_index, batch, W1, b1, W2, b2, Wh1, bh1, Wh2, bh2):
    src = edge_index[0]
    dst = edge_index[1]
    cnt = _sc_count(dst)
    y1, dinv = _tc_first(cnt, x, W1)
    agg1 = _sc_agg(src, dst, y1)
    y2 = _tc_mid(agg1, y1, dinv, b1, W2)
    agg2 = _sc_agg(src, dst, y2)
    return _tc_head(agg2, y2, dinv, b2, batch, Wh1, bh1, Wh2, bh2)


# speedup vs baseline: 1.4758x; 1.4758x over previous
import jax
import jax.numpy as jnp
from jax import lax
from jax.experimental import pallas as pl
from jax.experimental.pallas import tpu as pltpu
from jax.experimental.pallas import tpu_sc as plsc

N = 10000
E = 320000
F_IN = 128
H = 64
G = 128

NC = 2
NS = 16
NW = NC * NS
E_PER_TILE = E // NW
K = 80
CHUNKS = E_PER_TILE // K
N_PAD = 10240
ROWS_PER_TILE = N_PAD // NS
ZROWS = 128

CNT_W = 16


def _mesh():
    return plsc.VectorSubcoreMesh(
        core_axis_name="c", subcore_axis_name="s",
        num_cores=NC, num_subcores=NS)


_SC_PARAMS = pltpu.CompilerParams(use_tc_tiling_on_sc=False)


def _zero_fill(buf, width):
    def body(i, carry):
        for c in range(width // 16):
            buf[i, pl.ds(c * 16, 16)] = jnp.zeros((16,), jnp.float32)
        return carry
    lax.fori_loop(0, ZROWS, body, None)


def _zero_my_slice(zbuf, acc_sh, sid):
    for r in range(ROWS_PER_TILE // ZROWS):
        pltpu.sync_copy(zbuf, acc_sh.at[pl.ds(sid * ROWS_PER_TILE + r * ZROWS, ZROWS)])


NB = 5
GROUPS = CHUNKS // NB


def _sc_count_body(dst_hbm, out_hbm, didx_v, sdidx_v, ones_v, zbuf_v,
                   sem_i, sem_s, acc_sh):
    cid = lax.axis_index("c")
    sid = lax.axis_index("s")

    def fill_ones(i, carry):
        ones_v[i, :] = jnp.ones((16,), jnp.float32)
        return carry
    lax.fori_loop(0, K, fill_ones, None)
    _zero_fill(zbuf_v, CNT_W)
    _zero_my_slice(zbuf_v, acc_sh, sid)
    plsc.subcore_barrier()

    base = cid * (E // NC) + sid * E_PER_TILE

    def start_idx(j, b):
        pltpu.async_copy(dst_hbm.at[pl.ds(base + j * K, K)], didx_v.at[b],
                         sem_i.at[b])

    def wait_idx(b):
        pltpu.make_async_copy(dst_hbm.at[pl.ds(0, K)], didx_v.at[b],
                              sem_i.at[b]).wait()

    def wait_scat(b):
        pltpu.make_async_copy(ones_v, acc_sh.at[sdidx_v.at[b]],
                              sem_s.at[b]).wait()

    for b in range(NB):
        start_idx(b, b)

    def group(g, carry):
        for b in range(NB):
            j = g * NB + b
            wait_idx(b)

            @pl.when(g > 0)
            def _():
                wait_scat(b)
            for w in range(K // 16):
                sdidx_v[b, pl.ds(w * 16, 16)] = didx_v[b, pl.ds(w * 16, 16)]
            pltpu.async_copy(ones_v, acc_sh.at[sdidx_v.at[b]], sem_s.at[b],
                             add=True)

            @pl.when(j + NB < CHUNKS)
            def _():
                start_idx(j + NB, b)
        return carry
    lax.fori_loop(0, GROUPS, group, None)
    for b in range(NB):
        wait_scat(b)
    plsc.subcore_barrier()

    pltpu.sync_copy(
        acc_sh.at[pl.ds(sid * ROWS_PER_TILE, ROWS_PER_TILE)],
        out_hbm.at[pl.ds(cid * N_PAD + sid * ROWS_PER_TILE, ROWS_PER_TILE)])


_sc_count = pl.kernel(
    _sc_count_body,
    out_type=jax.ShapeDtypeStruct((NC * N_PAD, CNT_W), jnp.float32),
    mesh=_mesh(),
    scratch_types=[
        pltpu.VMEM((NB, K), jnp.int32),
        pltpu.VMEM((NB, K), jnp.int32),
        pltpu.VMEM((K, CNT_W), jnp.float32),
        pltpu.VMEM((ZROWS, CNT_W), jnp.float32),
        pltpu.SemaphoreType.DMA((NB,)),
        pltpu.SemaphoreType.DMA((NB,)),
        pltpu.VMEM_SHARED((N_PAD, CNT_W), jnp.float32),
    ],
    compiler_params=_SC_PARAMS,
)


def _sc_agg_body(src_hbm, dst_hbm, y_hbm, out_hbm,
                 sidx_v, didx_v, ssidx_v, sdidx_v, rows_v, zbuf_v,
                 sem_i, sem_g, sem_s, acc_sh):
    cid = lax.axis_index("c")
    sid = lax.axis_index("s")

    _zero_fill(zbuf_v, H)
    _zero_my_slice(zbuf_v, acc_sh, sid)
    plsc.subcore_barrier()

    base = cid * (E // NC) + sid * E_PER_TILE

    def start_idx(j, b):
        off = base + j * K
        pltpu.async_copy(src_hbm.at[pl.ds(off, K)], sidx_v.at[b], sem_i.at[b])
        pltpu.async_copy(dst_hbm.at[pl.ds(off, K)], didx_v.at[b], sem_i.at[b])

    def wait_idx(b):
        pltpu.make_async_copy(src_hbm.at[pl.ds(0, K)], sidx_v.at[b], sem_i.at[b]).wait()
        pltpu.make_async_copy(dst_hbm.at[pl.ds(0, K)], didx_v.at[b], sem_i.at[b]).wait()

    def wait_gather(b):
        pltpu.make_async_copy(y_hbm.at[ssidx_v.at[b]], rows_v.at[b],
                              sem_g.at[b]).wait()

    def wait_scat(b):
        pltpu.make_async_copy(rows_v.at[b], acc_sh.at[sdidx_v.at[b]],
                              sem_s.at[b]).wait()

    for b in range(NB):
        start_idx(b, b)

    def group(g, carry):
        for b in range(NB):
            j = g * NB + b
            wait_idx(b)

            @pl.when(g > 0)
            def _():
                wait_scat(b)
            for w in range(K // 16):
                ssidx_v[b, pl.ds(w * 16, 16)] = sidx_v[b, pl.ds(w * 16, 16)]
                sdidx_v[b, pl.ds(w * 16, 16)] = didx_v[b, pl.ds(w * 16, 16)]
            pltpu.async_copy(y_hbm.at[ssidx_v.at[b]], rows_v.at[b], sem_g.at[b])

            @pl.when(j + NB < CHUNKS)
            def _():
                start_idx(j + NB, b)
        for b in range(NB):
            wait_gather(b)
            pltpu.async_copy(rows_v.at[b], acc_sh.at[sdidx_v.at[b]],
                             sem_s.at[b], add=True)
        return carry
    lax.fori_loop(0, GROUPS, group, None)
    for b in range(NB):
        wait_scat(b)
    plsc.subcore_barrier()

    pltpu.sync_copy(
        acc_sh.at[pl.ds(sid * ROWS_PER_TILE, ROWS_PER_TILE)],
        out_hbm.at[pl.ds(cid * N_PAD + sid * ROWS_PER_TILE, ROWS_PER_TILE)])


_sc_agg = pl.kernel(
    _sc_agg_body,
    out_type=jax.ShapeDtypeStruct((NC * N_PAD, H), jnp.float32),
    mesh=_mesh(),
    scratch_types=[
        pltpu.VMEM((NB, K), jnp.int32),
        pltpu.VMEM((NB, K), jnp.int32),
        pltpu.VMEM((NB, K), jnp.int32),
        pltpu.VMEM((NB, K), jnp.int32),
        pltpu.VMEM((NB, K, H), jnp.float32),
        pltpu.VMEM((ZROWS, H), jnp.float32),
        pltpu.SemaphoreType.DMA((NB,)),
        pltpu.SemaphoreType.DMA((NB,)),
        pltpu.SemaphoreType.DMA((NB,)),
        pltpu.VMEM_SHARED((N_PAD, H), jnp.float32),
    ],
    compiler_params=_SC_PARAMS,
)


def _tc_first_body(cnt_ref, x_ref, w1_ref, y1_ref, dinv_ref):
    cnt = cnt_ref[...]
    deg = cnt[0:N, :] + cnt[N_PAD:N_PAD + N, :] + 1.0
    d16 = lax.rsqrt(deg)
    d64 = jnp.concatenate([d16] * (H // CNT_W), axis=1)
    t = jnp.dot(x_ref[...], w1_ref[...], preferred_element_type=jnp.float32)
    y1_ref[...] = t * d64
    dinv_ref[...] = d64


_tc_first = pl.pallas_call(
    _tc_first_body,
    out_shape=[
        jax.ShapeDtypeStruct((N, H), jnp.float32),
        jax.ShapeDtypeStruct((N, H), jnp.float32),
    ],
)


def _tc_mid_body(agg_ref, y1_ref, dinv_ref, b1_ref, w2_ref, y2_ref):
    agg = agg_ref[0:N, :] + agg_ref[N_PAD:N_PAD + N, :] + y1_ref[...]
    d = dinv_ref[...]
    h = jnp.maximum(d * agg + b1_ref[...], 0.0)
    y2_ref[...] = jnp.dot(h, w2_ref[...], preferred_element_type=jnp.float32) * d


_tc_mid = pl.pallas_call(
    _tc_mid_body,
    out_shape=jax.ShapeDtypeStruct((N, H), jnp.float32),
)


def _tc_head_body(agg_ref, y2_ref, dinv_ref, b2_ref, batch_ref,
                  wh1_ref, bh1_ref, wh2_ref, bh2_ref, out_ref):
    agg = agg_ref[0:N, :] + agg_ref[N_PAD:N_PAD + N, :] + y2_ref[...]
    h = jnp.maximum(dinv_ref[...] * agg + b2_ref[...], 0.0)
    seg = lax.broadcasted_iota(jnp.int32, (G, N), 0)
    mask = (seg == batch_ref[...][None, :]).astype(jnp.float32)
    sums = jnp.dot(mask, h, preferred_element_type=jnp.float32)
    cnt = jnp.sum(mask, axis=1, keepdims=True)
    pooled = sums / jnp.maximum(cnt, 1.0)
    z = jnp.maximum(
        jnp.dot(pooled, wh1_ref[...], preferred_element_type=jnp.float32)
        + bh1_ref[...], 0.0)
    out_ref[...] = (jnp.dot(z, wh2_ref[...], preferred_element_type=jnp.float32)
                    + bh2_ref[...])


_tc_head = pl.pallas_call(
    _tc_head_body,
    out_shape=jax.ShapeDtypeStruct((G, 2), jnp.float32),
)


def kernel(x, edge_index, batch, W1, b1, W2, b2, Wh1, bh1, Wh2, bh2):
    src = edge_index[0]
    dst = edge_index[1]
    cnt = _sc_count(dst)
    y1, dinv = _tc_first(cnt, x, W1)
    agg1 = _sc_agg(src, dst, y1)
    y2 = _tc_mid(agg1, y1, dinv, b1, W2)
    agg2 = _sc_agg(src, dst, y2)
    return _tc_head(agg2, y2, dinv, b2, batch, Wh1, bh1, Wh2, bh2)
